# baseline (speedup 1.0000x reference)
import jax
import jax.numpy as jnp
import numpy as np
from jax import lax
from jax.experimental import pallas as pl
from jax.experimental.pallas import tpu as pltpu
from jax.experimental.pallas import tpu_sc as plsc

D_MODEL = 64
MAX_SEQ = 200
BATCH = 1024
VOCAB = 1000000

NC = 2
NS = 16
NW = NC * NS
TOKENS = BATCH * MAX_SEQ
CHUNK = 100
N_CHUNKS_TOTAL = TOKENS // CHUNK
CHUNKS_PER_W = N_CHUNKS_TOTAL // NW
TOK_PER_W = TOKENS // NW
MROWS = TOK_PER_W // 16


def _pos_encoding():
    pos = np.arange(MAX_SEQ)[:, np.newaxis].astype(np.float64)
    i = np.arange(D_MODEL)[np.newaxis, :]
    rates = 1.0 / np.power(10000, 2 * (i // 2) / np.float32(D_MODEL))
    pr = pos * rates
    pr[:, 0::2] = np.sin(pr[:, 0::2])
    pr[:, 1::2] = np.cos(pr[:, 1::2])
    return pr.astype(np.float32)


_PE = _pos_encoding()


def _sc_body(table_hbm, idx2d_hbm, idxf_hbm, pe_hbm,
             x_hbm, mask_hbm,
             idx2d_v, idxf_v, pe_v,
             rows0_v, rows1_v, obuf0_v, obuf1_v, mbuf_v,
             gsem0, gsem1, wsem0, wsem1):
    cid = lax.axis_index("c")
    sid = lax.axis_index("s")
    wid = sid * NC + cid
    cbase = wid * CHUNKS_PER_W

    rows = (rows0_v, rows1_v)
    obuf = (obuf0_v, obuf1_v)
    gsem = (gsem0, gsem1)
    wsem = (wsem0, wsem1)

    pltpu.sync_copy(idx2d_hbm.at[pl.ds(cbase, CHUNKS_PER_W)], idx2d_v)
    pltpu.sync_copy(idxf_hbm.at[pl.ds(wid * MROWS, MROWS)], idxf_v)

    for b in range(2):
        pltpu.make_async_copy(
            table_hbm.at[idx2d_v.at[b]], rows[b], gsem[b]).start()

    pltpu.sync_copy(pe_hbm, pe_v)

    one = jnp.full((16,), 1.0, dtype=jnp.float32)
    zero = jnp.full((16,), 0.0, dtype=jnp.float32)

    def mask_body(i, carry):
        iv = idxf_v[i, :]
        mbuf_v[i, :] = jnp.where(iv != 0, one, zero)
        return carry

    lax.fori_loop(0, MROWS, mask_body, 0, unroll=4)
    pltpu.sync_copy(mbuf_v, mask_hbm.at[pl.ds(wid * MROWS, MROWS)])

    def process(c, b):
        pltpu.make_async_copy(
            table_hbm.at[idx2d_v.at[c]], rows[b], gsem[b]).wait()
        @pl.when(c >= 2)
        def _():
            pltpu.make_async_copy(
                obuf[b], x_hbm.at[cbase + c - 2], wsem[b]).wait()

        h = (c % 2) * CHUNK

        @plsc.parallel_loop(0, CHUNK)
        def row_body(r):
            pr = h + r
            for k in range(D_MODEL // 32):
                u = rows[b][r, pl.ds(16 * k, 16)]
                lo = plsc.bitcast(u << 16, jnp.float32)
                hi = plsc.bitcast(u & jnp.int32(-65536), jnp.float32)
                slo = pl.ds(16 * k, 16)
                shi = pl.ds(32 + 16 * k, 16)
                obuf[b][r, slo] = lo * 8.0 + pe_v[pr, slo]
                obuf[b][r, shi] = hi * 8.0 + pe_v[pr, shi]

        pltpu.make_async_copy(obuf[b], x_hbm.at[cbase + c], wsem[b]).start()

        @pl.when(c + 2 < CHUNKS_PER_W)
        def _():
            pltpu.make_async_copy(
                table_hbm.at[idx2d_v.at[c + 2]], rows[b], gsem[b]).start()

    def outer_body(g, carry):
        for b in range(2):
            process(2 * g + b, b)
        return carry

    lax.fori_loop(0, CHUNKS_PER_W // 2, outer_body, 0)

    for b in range(2):
        c = CHUNKS_PER_W - 2 + b
        pltpu.make_async_copy(obuf[b], x_hbm.at[cbase + c], wsem[b]).wait()


TBN = 32768


def _relayout_body(t_ref, p_ref, o_ref):
    o_ref[...] = lax.dot_general(
        t_ref[...].astype(jnp.bfloat16), p_ref[...].astype(jnp.bfloat16),
        (((0,), (0,)), ((), ())),
        preferred_element_type=jnp.float32).astype(jnp.bfloat16)


def _relayout_table(emb_table):
    tbl_t = emb_table.T
    proj_np = np.zeros((D_MODEL, 128), np.float32)
    for i in range(32):
        proj_np[i, 2 * i] = 1.0
        proj_np[32 + i, 2 * i + 1] = 1.0
    proj = jnp.asarray(proj_np)
    ncols = tbl_t.shape[1]
    grid = (ncols + TBN - 1) // TBN
    return pl.pallas_call(
        _relayout_body,
        grid=(grid,),
        in_specs=[pl.BlockSpec((D_MODEL, TBN), lambda i: (0, i)),
                  pl.BlockSpec((D_MODEL, 128), lambda i: (0, 0))],
        out_specs=pl.BlockSpec((TBN, 128), lambda i: (i, 0)),
        out_shape=jax.ShapeDtypeStruct((VOCAB, 128), jnp.bfloat16),
    )(tbl_t, proj)


@jax.jit
def _pre_layer(inputs, emb_table):
    idx2d = inputs.reshape(N_CHUNKS_TOTAL, CHUNK)
    idxf = inputs.reshape(TOKENS // 16, 16)
    pe = jnp.asarray(_PE)
    tbl = jax.lax.bitcast_convert_type(
        _relayout_table(emb_table).reshape(VOCAB, D_MODEL, 2), jnp.int32)

    mesh = plsc.VectorSubcoreMesh(core_axis_name="c", subcore_axis_name="s")
    x_flat, mask_flat = pl.kernel(
        _sc_body,
        out_type=(
            jax.ShapeDtypeStruct((N_CHUNKS_TOTAL, CHUNK, D_MODEL),
                                 jnp.float32),
            jax.ShapeDtypeStruct((TOKENS // 16, 16), jnp.float32),
        ),
        mesh=mesh,
        scratch_types=[
            pltpu.VMEM((CHUNKS_PER_W, CHUNK), jnp.int32),
            pltpu.VMEM((MROWS, 16), jnp.int32),
            pltpu.VMEM((MAX_SEQ, D_MODEL), jnp.float32),
            pltpu.VMEM((CHUNK, D_MODEL), jnp.int32),
            pltpu.VMEM((CHUNK, D_MODEL), jnp.int32),
            pltpu.VMEM((CHUNK, D_MODEL), jnp.float32),
            pltpu.VMEM((CHUNK, D_MODEL), jnp.float32),
            pltpu.VMEM((MROWS, 16), jnp.float32),
            pltpu.SemaphoreType.DMA,
            pltpu.SemaphoreType.DMA,
            pltpu.SemaphoreType.DMA,
            pltpu.SemaphoreType.DMA,
        ],
        compiler_params=pltpu.CompilerParams(use_tc_tiling_on_sc=False,
                                             needs_layout_passes=False),
    )(tbl, idx2d, idxf, pe)

    x = x_flat.reshape(BATCH, MAX_SEQ, D_MODEL)
    padding_mask = mask_flat.reshape(BATCH, 1, 1, MAX_SEQ)
    return x, padding_mask


def kernel(inputs, emb_table, training=False):
    del training
    return _pre_layer(inputs, emb_table)


# speedup vs baseline: 7.9912x; 7.9912x over previous
import jax
import jax.numpy as jnp
import numpy as np
from jax import lax
from jax.experimental import pallas as pl
from jax.experimental.pallas import tpu as pltpu
from jax.experimental.pallas import tpu_sc as plsc

D_MODEL = 64
MAX_SEQ = 200
BATCH = 1024
VOCAB = 1000000

NC = 2
NS = 16
NW = NC * NS
TOKENS = BATCH * MAX_SEQ
CHUNK = 100
N_CHUNKS_TOTAL = TOKENS // CHUNK
CHUNKS_PER_W = N_CHUNKS_TOTAL // NW
TOK_PER_W = TOKENS // NW
MROWS = TOK_PER_W // 16


def _pos_encoding():
    pos = np.arange(MAX_SEQ)[:, np.newaxis].astype(np.float64)
    i = np.arange(D_MODEL)[np.newaxis, :]
    rates = 1.0 / np.power(10000, 2 * (i // 2) / np.float32(D_MODEL))
    pr = pos * rates
    pr[:, 0::2] = np.sin(pr[:, 0::2])
    pr[:, 1::2] = np.cos(pr[:, 1::2])
    return pr.astype(np.float32)


_PE = _pos_encoding()


def _sc_body(table_hbm, idx2d_hbm, idxf_hbm, pe_hbm,
             x_hbm, mask_hbm,
             idx2d_v, idxf_v, pe_v,
             rows0_v, rows1_v, obuf0_v, obuf1_v, mbuf_v,
             gsem0, gsem1, wsem0, wsem1):
    cid = lax.axis_index("c")
    sid = lax.axis_index("s")
    wid = sid * NC + cid
    cbase = wid * CHUNKS_PER_W

    rows = (rows0_v, rows1_v)
    obuf = (obuf0_v, obuf1_v)
    gsem = (gsem0, gsem1)
    wsem = (wsem0, wsem1)

    pltpu.sync_copy(idx2d_hbm.at[pl.ds(cbase, CHUNKS_PER_W)], idx2d_v)
    pltpu.sync_copy(idxf_hbm.at[pl.ds(wid * MROWS, MROWS)], idxf_v)

    for b in range(2):
        pltpu.make_async_copy(
            table_hbm.at[idx2d_v.at[b]], rows[b], gsem[b]).start()

    pltpu.sync_copy(pe_hbm, pe_v)

    one = jnp.full((16,), 1.0, dtype=jnp.float32)
    zero = jnp.full((16,), 0.0, dtype=jnp.float32)

    def mask_body(i, carry):
        iv = idxf_v[i, :]
        mbuf_v[i, :] = jnp.where(iv != 0, one, zero)
        return carry

    lax.fori_loop(0, MROWS, mask_body, 0, unroll=4)
    pltpu.sync_copy(mbuf_v, mask_hbm.at[pl.ds(wid * MROWS, MROWS)])

    def process(c, b):
        pltpu.make_async_copy(
            table_hbm.at[idx2d_v.at[c]], rows[b], gsem[b]).wait()
        @pl.when(c >= 2)
        def _():
            pltpu.make_async_copy(
                obuf[b], x_hbm.at[cbase + c - 2], wsem[b]).wait()

        h = (c % 2) * CHUNK

        @plsc.parallel_loop(0, CHUNK)
        def row_body(r):
            pr = h + r
            for j in range(D_MODEL // 16):
                sl = pl.ds(j * 16, 16)
                obuf[b][r, sl] = rows[b][r, sl] * 8.0 + pe_v[pr, sl]

        pltpu.make_async_copy(obuf[b], x_hbm.at[cbase + c], wsem[b]).start()

        @pl.when(c + 2 < CHUNKS_PER_W)
        def _():
            pltpu.make_async_copy(
                table_hbm.at[idx2d_v.at[c + 2]], rows[b], gsem[b]).start()

    def outer_body(g, carry):
        for b in range(2):
            process(2 * g + b, b)
        return carry

    lax.fori_loop(0, CHUNKS_PER_W // 2, outer_body, 0)

    for b in range(2):
        c = CHUNKS_PER_W - 2 + b
        pltpu.make_async_copy(obuf[b], x_hbm.at[cbase + c], wsem[b]).wait()


TBN = 32768


def _relayout_body(t_ref, p_ref, o_ref):
    o_ref[...] = lax.dot_general(
        t_ref[...].astype(jnp.bfloat16), p_ref[...].astype(jnp.bfloat16),
        (((0,), (0,)), ((), ())),
        preferred_element_type=jnp.float32)


def _relayout_table(emb_table):
    tbl_t = emb_table.T
    proj = jnp.concatenate(
        [jnp.eye(D_MODEL, dtype=jnp.float32),
         jnp.zeros((D_MODEL, 128 - D_MODEL), jnp.float32)], axis=1)
    ncols = tbl_t.shape[1]
    grid = (ncols + TBN - 1) // TBN
    return pl.pallas_call(
        _relayout_body,
        grid=(grid,),
        in_specs=[pl.BlockSpec((D_MODEL, TBN), lambda i: (0, i)),
                  pl.BlockSpec((D_MODEL, 128), lambda i: (0, 0))],
        out_specs=pl.BlockSpec((TBN, 128), lambda i: (i, 0)),
        out_shape=jax.ShapeDtypeStruct((VOCAB, 128), jnp.float32),
    )(tbl_t, proj)


@jax.jit
def _pre_layer(inputs, emb_table):
    idx2d = inputs.reshape(N_CHUNKS_TOTAL, CHUNK)
    idxf = inputs.reshape(TOKENS // 16, 16)
    pe = jnp.asarray(_PE)
    tbl = _relayout_table(emb_table)

    mesh = plsc.VectorSubcoreMesh(core_axis_name="c", subcore_axis_name="s")
    x_flat, mask_flat = pl.kernel(
        _sc_body,
        out_type=(
            jax.ShapeDtypeStruct((N_CHUNKS_TOTAL, CHUNK, D_MODEL),
                                 jnp.float32),
            jax.ShapeDtypeStruct((TOKENS // 16, 16), jnp.float32),
        ),
        mesh=mesh,
        scratch_types=[
            pltpu.VMEM((CHUNKS_PER_W, CHUNK), jnp.int32),
            pltpu.VMEM((MROWS, 16), jnp.int32),
            pltpu.VMEM((MAX_SEQ, D_MODEL), jnp.float32),
            pltpu.VMEM((CHUNK, 128), jnp.float32),
            pltpu.VMEM((CHUNK, 128), jnp.float32),
            pltpu.VMEM((CHUNK, D_MODEL), jnp.float32),
            pltpu.VMEM((CHUNK, D_MODEL), jnp.float32),
            pltpu.VMEM((MROWS, 16), jnp.float32),
            pltpu.SemaphoreType.DMA,
            pltpu.SemaphoreType.DMA,
            pltpu.SemaphoreType.DMA,
            pltpu.SemaphoreType.DMA,
        ],
        compiler_params=pltpu.CompilerParams(use_tc_tiling_on_sc=False),
    )(tbl, idx2d, idxf, pe)

    x = x_flat.reshape(BATCH, MAX_SEQ, D_MODEL)
    padding_mask = mask_flat.reshape(BATCH, 1, 1, MAX_SEQ)
    return x, padding_mask


def kernel(inputs, emb_table, training=False):
    del training
    return _pre_layer(inputs, emb_table)
